# Initial kernel scaffold; baseline (speedup 1.0000x reference)
import jax
import jax.numpy as jnp
from jax.experimental import pallas as pl


def kernel(idx, W):
    raise NotImplementedError("write your pallas kernel here")


# speedup vs baseline: 4.5643x; 4.5643x over previous
import functools

import jax
import jax.numpy as jnp
from jax import lax
from jax.experimental import pallas as pl
from jax.experimental.pallas import tpu as pltpu
from jax.experimental.pallas import tpu_sc as plsc

_NC = 2
_NS = 16
_NW = _NC * _NS


def _build(B, D):
    PW = B // _NW
    C = 128
    K = 5
    GR = C * K
    NG = PW // GR
    assert PW % GR == 0 and B % _NW == 0

    mesh = plsc.VectorSubcoreMesh(core_axis_name="c", subcore_axis_name="s")

    @functools.partial(
        pl.kernel,
        out_type=jax.ShapeDtypeStruct((B, D), jnp.float32),
        mesh=mesh,
        scratch_types=[
            pltpu.VMEM((PW,), jnp.int32),
            pltpu.VMEM((GR, D), jnp.float32),
            pltpu.SemaphoreType.DMA,
        ],
        compiler_params=pltpu.CompilerParams(use_tc_tiling_on_sc=False),
    )
    def emb(idx_hbm, w_hbm, out_hbm, idx_v, rows_v, gsem):
        wid = lax.axis_index("s") * _NC + lax.axis_index("c")
        base = wid * PW
        pltpu.sync_copy(idx_hbm.at[pl.ds(base, PW)], idx_v)

        def group(g, carry):
            off = g * GR
            copies = [
                pltpu.async_copy(
                    w_hbm.at[idx_v.at[pl.ds(off + j * C, C)]],
                    rows_v.at[pl.ds(j * C, C)],
                    gsem,
                )
                for j in range(K)
            ]
            for c in copies:
                c.wait()
            pltpu.sync_copy(rows_v, out_hbm.at[pl.ds(base + off, GR)])
            return carry

        lax.fori_loop(0, NG, group, None)

    return emb


def kernel(idx, W):
    B, T = idx.shape
    D = W.shape[1]
    flat = idx.reshape(B * T).astype(jnp.int32)
    out = _build(B * T, D)(flat, W)
    return out.reshape(B, T, D)


# speedup vs baseline: 4.6118x; 1.0104x over previous
import functools

import jax
import jax.numpy as jnp
from jax import lax
from jax.experimental import pallas as pl
from jax.experimental.pallas import tpu as pltpu
from jax.experimental.pallas import tpu_sc as plsc

_NC = 2
_NS = 16
_NW = _NC * _NS


def _build(B, D):
    PW = B // _NW
    C = 128
    K = 5
    GR = C * K
    NG = PW // GR
    assert PW % GR == 0 and B % _NW == 0

    mesh = plsc.VectorSubcoreMesh(core_axis_name="c", subcore_axis_name="s")

    @functools.partial(
        pl.kernel,
        out_type=jax.ShapeDtypeStruct((B, D), jnp.float32),
        mesh=mesh,
        scratch_types=[
            pltpu.VMEM((PW,), jnp.int32),
            pltpu.VMEM((2, GR, D), jnp.float32),
            pltpu.SemaphoreType.DMA,
            pltpu.SemaphoreType.DMA,
        ],
        compiler_params=pltpu.CompilerParams(use_tc_tiling_on_sc=False),
    )
    def emb(idx_hbm, w_hbm, out_hbm, idx_v, rows_v, gsem, osem):
        wid = lax.axis_index("s") * _NC + lax.axis_index("c")
        base = wid * PW
        pltpu.sync_copy(idx_hbm.at[pl.ds(base, PW)], idx_v)

        def fire(g, buf):
            off = g * GR
            for j in range(K):
                pltpu.async_copy(
                    w_hbm.at[idx_v.at[pl.ds(off + j * C, C)]],
                    buf.at[pl.ds(j * C, C)],
                    gsem,
                )

        def drain_gathers():
            for j in range(K):
                pltpu.make_async_copy(
                    w_hbm.at[pl.ds(0, C)],
                    rows_v.at[0].at[pl.ds(j * C, C)],
                    gsem,
                ).wait()

        def drain_write():
            pltpu.make_async_copy(
                rows_v.at[0], out_hbm.at[pl.ds(0, GR)], osem
            ).wait()

        fire(0, rows_v.at[0])

        def group(g, carry):
            p = lax.rem(g, 2)
            buf = rows_v.at[p]
            other = rows_v.at[1 - p]
            drain_gathers()

            @pl.when(g >= 1)
            def _():
                drain_write()

            @pl.when(g + 1 < NG)
            def _():
                fire(g + 1, other)

            pltpu.async_copy(buf, out_hbm.at[pl.ds(base + g * GR, GR)], osem)
            return carry

        lax.fori_loop(0, NG, group, None)
        drain_write()

    return emb


def kernel(idx, W):
    B, T = idx.shape
    D = W.shape[1]
    flat = idx.reshape(B * T).astype(jnp.int32)
    out = _build(B * T, D)(flat, W)
    return out.reshape(B, T, D)


# speedup vs baseline: 6.2695x; 1.3595x over previous
import functools

import jax
import jax.numpy as jnp
from jax import lax
from jax.experimental import pallas as pl
from jax.experimental.pallas import tpu as pltpu
from jax.experimental.pallas import tpu_sc as plsc

_NC = 2
_NS = 16
_NW = _NC * _NS
_L = 16


def _build(T, B, D, V):
    DPT = D // _NW

    mesh = plsc.VectorSubcoreMesh(core_axis_name="c", subcore_axis_name="s")

    @functools.partial(
        pl.kernel,
        out_type=jax.ShapeDtypeStruct((T, D, B), jnp.float32),
        mesh=mesh,
        scratch_types=[
            pltpu.VMEM((V,), jnp.float32),
            pltpu.VMEM((B,), jnp.int32),
            pltpu.VMEM((B,), jnp.float32),
        ],
        compiler_params=pltpu.CompilerParams(needs_layout_passes=False),
    )
    def emb(idxt_hbm, wt_hbm, out_hbm, row_v, idx_v, out_v):
        wid = lax.axis_index("s") * _NC + lax.axis_index("c")

        for di in range(DPT):
            d = wid + di * _NW
            pltpu.sync_copy(wt_hbm.at[d], row_v)

            def per_t(t, carry):
                pltpu.sync_copy(idxt_hbm.at[t], idx_v)

                @plsc.parallel_loop(0, B, step=_L, unroll=8)
                def _(i):
                    v = idx_v[pl.ds(i, _L)]
                    out_v[pl.ds(i, _L)] = plsc.load_gather(row_v, [v])

                pltpu.sync_copy(out_v, out_hbm.at[t].at[d])
                return carry

            lax.fori_loop(0, T, per_t, None)

    return emb


def kernel(idx, W):
    B, T = idx.shape
    V, D = W.shape
    idxt = idx.T.astype(jnp.int32)
    wt = W.T
    out3 = _build(T, B, D, V)(idxt, wt)
    return out3.transpose(2, 0, 1)


# speedup vs baseline: 8.4558x; 1.3487x over previous
import functools

import jax
import jax.numpy as jnp
from jax import lax
from jax.experimental import pallas as pl
from jax.experimental.pallas import tpu as pltpu
from jax.experimental.pallas import tpu_sc as plsc

_NC = 2
_NS = 16
_NW = _NC * _NS
_L = 16


def _build(T, B, D, V):
    DPT = D // _NW

    mesh = plsc.VectorSubcoreMesh(core_axis_name="c", subcore_axis_name="s")

    @functools.partial(
        pl.kernel,
        out_type=jax.ShapeDtypeStruct((T, D, B), jnp.float32),
        mesh=mesh,
        scratch_types=[
            pltpu.VMEM((V,), jnp.float32),
            pltpu.VMEM((2 * B,), jnp.int32),
            pltpu.VMEM((2 * B,), jnp.float32),
            pltpu.SemaphoreType.DMA,
            pltpu.SemaphoreType.DMA,
        ],
        compiler_params=pltpu.CompilerParams(needs_layout_passes=False),
    )
    def emb(idxt_hbm, wt_hbm, out_hbm, row_v, idx2_v, out2_v, isem, osem):
        wid = lax.axis_index("s") * _NC + lax.axis_index("c")

        def drain_idx():
            pltpu.make_async_copy(idxt_hbm.at[0], idx2_v.at[pl.ds(0, B)], isem).wait()

        def drain_out():
            pltpu.make_async_copy(out2_v.at[pl.ds(0, B)], out_hbm.at[0].at[0], osem).wait()

        for di in range(DPT):
            d = wid + di * _NW
            pltpu.sync_copy(wt_hbm.at[d], row_v)
            pltpu.async_copy(idxt_hbm.at[0], idx2_v.at[pl.ds(0, B)], isem)

            def per_t(t, carry):
                pb = lax.rem(t, 2) * B
                qb = B - pb
                drain_idx()

                @pl.when(t + 1 < T)
                def _():
                    pltpu.async_copy(
                        idxt_hbm.at[t + 1], idx2_v.at[pl.ds(qb, B)], isem
                    )

                @pl.when(t >= 2)
                def _():
                    drain_out()

                @plsc.parallel_loop(0, B, step=_L, unroll=8)
                def _(i):
                    out2_v[pl.ds(pb + i, _L)] = plsc.load_gather(
                        row_v, [idx2_v[pl.ds(pb + i, _L)]]
                    )

                pltpu.async_copy(
                    out2_v.at[pl.ds(pb, B)], out_hbm.at[t].at[d], osem
                )
                return carry

            lax.fori_loop(0, T, per_t, None)
            drain_out()
            drain_out()

    return emb


def kernel(idx, W):
    B, T = idx.shape
    V, D = W.shape
    idxt = idx.T.astype(jnp.int32)
    wt = W.T
    out3 = _build(T, B, D, V)(idxt, wt)
    return out3.transpose(2, 0, 1)


# speedup vs baseline: 11.1831x; 1.3225x over previous
import functools

import jax
import jax.numpy as jnp
from jax import lax
from jax.experimental import pallas as pl
from jax.experimental.pallas import tpu as pltpu
from jax.experimental.pallas import tpu_sc as plsc

_NC = 2
_NS = 16
_NW = _NC * _NS
_L = 16


def _build(T, B, D, V):
    DPT = D // _NW

    mesh = plsc.VectorSubcoreMesh(core_axis_name="c", subcore_axis_name="s")

    @functools.partial(
        pl.kernel,
        out_type=jax.ShapeDtypeStruct((T, D, B), jnp.float32),
        mesh=mesh,
        scratch_types=[
            pltpu.VMEM((V,), jnp.float32),
            pltpu.VMEM((3 * B,), jnp.int32),
            pltpu.VMEM((2 * B,), jnp.float32),
            pltpu.SemaphoreType.DMA,
            pltpu.SemaphoreType.DMA,
        ],
        compiler_params=pltpu.CompilerParams(needs_layout_passes=False),
    )
    def emb(idxt_hbm, wt_hbm, out_hbm, row_v, idx2_v, out2_v, isem, osem):
        wid = lax.axis_index("s") * _NC + lax.axis_index("c")

        def drain_idx():
            pltpu.make_async_copy(idxt_hbm.at[0], idx2_v.at[pl.ds(0, B)], isem).wait()

        def drain_out():
            pltpu.make_async_copy(out2_v.at[pl.ds(0, B)], out_hbm.at[0].at[0], osem).wait()

        for di in range(DPT):
            d = wid + di * _NW
            pltpu.sync_copy(wt_hbm.at[d], row_v)
            pltpu.async_copy(idxt_hbm.at[0], idx2_v.at[pl.ds(0, B)], isem)
            pltpu.async_copy(idxt_hbm.at[1], idx2_v.at[pl.ds(B, B)], isem)

            def per_t(t, carry):
                ib = lax.rem(t, 3) * B
                pb = lax.rem(t, 2) * B

                @pl.when(t + 2 < T)
                def _():
                    pltpu.async_copy(
                        idxt_hbm.at[t + 2],
                        idx2_v.at[pl.ds(lax.rem(t + 2, 3) * B, B)],
                        isem,
                    )

                drain_idx()

                @pl.when(t >= 2)
                def _():
                    drain_out()

                @plsc.parallel_loop(0, B, step=_L, unroll=16)
                def _(i):
                    out2_v[pl.ds(pb + i, _L)] = plsc.load_gather(
                        row_v, [idx2_v[pl.ds(ib + i, _L)]]
                    )

                pltpu.async_copy(
                    out2_v.at[pl.ds(pb, B)], out_hbm.at[t].at[d], osem
                )
                return carry

            lax.fori_loop(0, T, per_t, None)
            drain_out()
            drain_out()

    return emb


def kernel(idx, W):
    B, T = idx.shape
    V, D = W.shape
    idxt = idx.T.astype(jnp.int32)
    wt = W.T
    out3 = _build(T, B, D, V)(idxt, wt)
    return out3.transpose(2, 0, 1)


# speedup vs baseline: 13.8281x; 1.2365x over previous
import functools

import jax
import jax.numpy as jnp
from jax import lax
from jax.experimental import pallas as pl
from jax.experimental.pallas import tpu as pltpu
from jax.experimental.pallas import tpu_sc as plsc

_NC = 2
_NS = 16
_NW = _NC * _NS
_L = 16


def _build(T, B, D, V):
    DPT = D // _NW

    spmem_words = 2097151
    tile_words = V + 3 * B + 2 * B
    free_words = spmem_words - _NS * tile_words - 8192
    T_SH = min(T, free_words // B)

    mesh = plsc.VectorSubcoreMesh(core_axis_name="c", subcore_axis_name="s")

    @functools.partial(
        pl.kernel,
        out_type=jax.ShapeDtypeStruct((T, D, B), jnp.float32),
        mesh=mesh,
        scratch_types=[
            pltpu.VMEM((V,), jnp.float32),
            pltpu.VMEM((3 * B,), jnp.int32),
            pltpu.VMEM((2 * B,), jnp.float32),
            pltpu.VMEM_SHARED((T_SH * B,), jnp.int32),
            pltpu.SemaphoreType.DMA,
            pltpu.SemaphoreType.DMA,
        ],
        compiler_params=pltpu.CompilerParams(needs_layout_passes=False),
    )
    def emb(idxt_hbm, wt_hbm, out_hbm, row_v, idx2_v, out2_v, idx_sh, isem, osem):
        wid = lax.axis_index("s") * _NC + lax.axis_index("c")
        sid = lax.axis_index("s")

        chunk = (T_SH * B) // _NS
        pltpu.sync_copy(
            idxt_hbm.at[pl.ds(sid * chunk, chunk)],
            idx_sh.at[pl.ds(sid * chunk, chunk)],
        )
        plsc.subcore_barrier()

        def drain_idx():
            pltpu.make_async_copy(
                idx_sh.at[pl.ds(0, B)], idx2_v.at[pl.ds(0, B)], isem
            ).wait()

        def drain_out():
            pltpu.make_async_copy(out2_v.at[pl.ds(0, B)], out_hbm.at[0].at[0], osem).wait()

        for di in range(DPT):
            d = wid + di * _NW
            pltpu.sync_copy(wt_hbm.at[d], row_v)
            pltpu.async_copy(idx_sh.at[pl.ds(0, B)], idx2_v.at[pl.ds(0, B)], isem)
            pltpu.async_copy(idx_sh.at[pl.ds(B, B)], idx2_v.at[pl.ds(B, B)], isem)

            def per_t(t, carry):
                ib = lax.rem(t, 3) * B
                pb = lax.rem(t, 2) * B

                @pl.when(jnp.logical_and(t + 2 >= T_SH, t + 2 < T))
                def _():
                    pltpu.async_copy(
                        idxt_hbm.at[pl.ds((t + 2) * B, B)],
                        idx2_v.at[pl.ds(lax.rem(t + 2, 3) * B, B)],
                        isem,
                    )

                @pl.when(t + 2 < T_SH)
                def _():
                    pltpu.async_copy(
                        idx_sh.at[pl.ds((t + 2) * B, B)],
                        idx2_v.at[pl.ds(lax.rem(t + 2, 3) * B, B)],
                        isem,
                    )

                drain_idx()

                @pl.when(t >= 2)
                def _():
                    drain_out()

                @plsc.parallel_loop(0, B, step=_L, unroll=16)
                def _(i):
                    out2_v[pl.ds(pb + i, _L)] = plsc.load_gather(
                        row_v, [idx2_v[pl.ds(ib + i, _L)]]
                    )

                pltpu.async_copy(
                    out2_v.at[pl.ds(pb, B)], out_hbm.at[t].at[d], osem
                )
                return carry

            lax.fori_loop(0, T, per_t, None)
            drain_out()
            drain_out()

    return emb


def kernel(idx, W):
    B, T = idx.shape
    V, D = W.shape
    idxt = idx.T.astype(jnp.int32).reshape(-1)
    wt = W.T
    out3 = _build(T, B, D, V)(idxt, wt)
    return out3.transpose(2, 0, 1)


# speedup vs baseline: 14.0820x; 1.0184x over previous
import functools

import jax
import jax.numpy as jnp
from jax import lax
from jax.experimental import pallas as pl
from jax.experimental.pallas import tpu as pltpu
from jax.experimental.pallas import tpu_sc as plsc

_NC = 2
_NS = 16
_NW = _NC * _NS
_L = 16


def _build(T, B, D, V):
    DPT = D // _NW

    spmem_words = 2097151
    tile_words = V + 3 * B + 2 * B
    free_words = spmem_words - _NS * tile_words - 8192
    T_SH = min(T, free_words // B)

    mesh = plsc.VectorSubcoreMesh(core_axis_name="c", subcore_axis_name="s")

    @functools.partial(
        pl.kernel,
        out_type=jax.ShapeDtypeStruct((T, D, B), jnp.float32),
        mesh=mesh,
        scratch_types=[
            pltpu.VMEM((V,), jnp.float32),
            pltpu.VMEM((3 * B,), jnp.int32),
            pltpu.VMEM((2 * B,), jnp.float32),
            pltpu.VMEM_SHARED((T_SH * B,), jnp.int32),
            pltpu.SemaphoreType.DMA,
            pltpu.SemaphoreType.DMA,
            pltpu.SemaphoreType.DMA,
        ],
        compiler_params=pltpu.CompilerParams(needs_layout_passes=False),
    )
    def emb(idxt_hbm, wt_hbm, out_hbm, row_v, idx2_v, out2_v, idx_sh, isem, osem, wsem):
        wid = lax.axis_index("s") * _NC + lax.axis_index("c")
        sid = lax.axis_index("s")

        pltpu.async_copy(wt_hbm.at[wid], row_v, wsem)

        chunk = (T_SH * B) // _NS
        pltpu.sync_copy(
            idxt_hbm.at[pl.ds(sid * chunk, chunk)],
            idx_sh.at[pl.ds(sid * chunk, chunk)],
        )
        plsc.subcore_barrier()

        def drain_idx():
            pltpu.make_async_copy(
                idx_sh.at[pl.ds(0, B)], idx2_v.at[pl.ds(0, B)], isem
            ).wait()

        def drain_out():
            pltpu.make_async_copy(out2_v.at[pl.ds(0, B)], out_hbm.at[0].at[0], osem).wait()

        for di in range(DPT):
            d = wid + di * _NW
            pltpu.async_copy(idx_sh.at[pl.ds(0, B)], idx2_v.at[pl.ds(0, B)], isem)
            pltpu.async_copy(idx_sh.at[pl.ds(B, B)], idx2_v.at[pl.ds(B, B)], isem)
            pltpu.make_async_copy(wt_hbm.at[d], row_v, wsem).wait()

            def per_t(t, carry):
                ib = lax.rem(t, 3) * B
                pb = lax.rem(t, 2) * B

                @pl.when(jnp.logical_and(t + 2 >= T_SH, t + 2 < T))
                def _():
                    pltpu.async_copy(
                        idxt_hbm.at[pl.ds((t + 2) * B, B)],
                        idx2_v.at[pl.ds(lax.rem(t + 2, 3) * B, B)],
                        isem,
                    )

                @pl.when(t + 2 < T_SH)
                def _():
                    pltpu.async_copy(
                        idx_sh.at[pl.ds((t + 2) * B, B)],
                        idx2_v.at[pl.ds(lax.rem(t + 2, 3) * B, B)],
                        isem,
                    )

                drain_idx()

                @pl.when(t >= 2)
                def _():
                    drain_out()

                @plsc.parallel_loop(0, B, step=_L, unroll=16)
                def _(i):
                    out2_v[pl.ds(pb + i, _L)] = plsc.load_gather(
                        row_v, [idx2_v[pl.ds(ib + i, _L)]]
                    )

                pltpu.async_copy(
                    out2_v.at[pl.ds(pb, B)], out_hbm.at[t].at[d], osem
                )
                return carry

            lax.fori_loop(0, T, per_t, None)
            if di + 1 < DPT:
                pltpu.async_copy(wt_hbm.at[wid + (di + 1) * _NW], row_v, wsem)
            drain_out()
            drain_out()

    return emb


def kernel(idx, W):
    B, T = idx.shape
    V, D = W.shape
    idxt = idx.T.astype(jnp.int32).reshape(-1)
    wt = W.T
    out3 = _build(T, B, D, V)(idxt, wt)
    return out3.transpose(2, 0, 1)


# speedup vs baseline: 14.2220x; 1.0099x over previous
import functools

import jax
import jax.numpy as jnp
from jax import lax
from jax.experimental import pallas as pl
from jax.experimental.pallas import tpu as pltpu
from jax.experimental.pallas import tpu_sc as plsc

_NC = 2
_NS = 16
_NW = _NC * _NS
_L = 16


def _build(T, B, D, V):
    DPT = D // _NW

    spmem_words = 2097151
    tile_words = V + 3 * B + 2 * B
    free_words = spmem_words - _NS * tile_words - 8192
    T_SH = min(T, free_words // B)

    mesh = plsc.VectorSubcoreMesh(core_axis_name="c", subcore_axis_name="s")

    @functools.partial(
        pl.kernel,
        out_type=jax.ShapeDtypeStruct((T, D, B), jnp.float32),
        mesh=mesh,
        scratch_types=[
            pltpu.VMEM((V,), jnp.float32),
            pltpu.VMEM((3 * B,), jnp.int32),
            pltpu.VMEM((2 * B,), jnp.float32),
            pltpu.VMEM_SHARED((T_SH * B,), jnp.int32),
            pltpu.SemaphoreType.DMA,
            pltpu.SemaphoreType.DMA,
            pltpu.SemaphoreType.DMA,
        ],
        compiler_params=pltpu.CompilerParams(needs_layout_passes=False),
    )
    def emb(idxt_hbm, wt_hbm, out_hbm, row_v, idx2_v, out2_v, idx_sh, isem, osem, wsem):
        wid = lax.axis_index("s") * _NC + lax.axis_index("c")
        sid = lax.axis_index("s")

        pltpu.async_copy(wt_hbm.at[wid], row_v, wsem)

        chunk = (T_SH * B) // _NS
        pltpu.sync_copy(
            idxt_hbm.at[pl.ds(sid * chunk, chunk)],
            idx_sh.at[pl.ds(sid * chunk, chunk)],
        )
        plsc.subcore_barrier()

        def drain_idx():
            pltpu.make_async_copy(
                idx_sh.at[pl.ds(0, B)], idx2_v.at[pl.ds(0, B)], isem
            ).wait()

        def drain_out():
            pltpu.make_async_copy(out2_v.at[pl.ds(0, B)], out_hbm.at[0].at[0], osem).wait()

        for di in range(DPT):
            d = wid + di * _NW
            pltpu.async_copy(idx_sh.at[pl.ds(0, B)], idx2_v.at[pl.ds(0, B)], isem)
            pltpu.async_copy(idx_sh.at[pl.ds(B, B)], idx2_v.at[pl.ds(B, B)], isem)
            pltpu.make_async_copy(wt_hbm.at[d], row_v, wsem).wait()

            def per_t(t, carry):
                iu, ju, pu = carry
                ib = iu * B
                jb = ju * B
                pb = pu * B

                @pl.when(jnp.logical_and(t + 2 >= T_SH, t + 2 < T))
                def _():
                    pltpu.async_copy(
                        idxt_hbm.at[pl.ds((t + 2) * B, B)],
                        idx2_v.at[pl.ds(jb, B)],
                        isem,
                    )

                @pl.when(t + 2 < T_SH)
                def _():
                    pltpu.async_copy(
                        idx_sh.at[pl.ds((t + 2) * B, B)],
                        idx2_v.at[pl.ds(jb, B)],
                        isem,
                    )

                drain_idx()

                @pl.when(t >= 2)
                def _():
                    drain_out()

                @plsc.parallel_loop(0, B, step=_L, unroll=32)
                def _(i):
                    out2_v[pl.ds(pb + i, _L)] = plsc.load_gather(
                        row_v, [idx2_v[pl.ds(ib + i, _L)]]
                    )

                pltpu.async_copy(
                    out2_v.at[pl.ds(pb, B)], out_hbm.at[t].at[d], osem
                )
                return (3 - iu - ju, iu, 1 - pu)

            lax.fori_loop(0, T, per_t, (0, 2, 0))
            if di + 1 < DPT:
                pltpu.async_copy(wt_hbm.at[wid + (di + 1) * _NW], row_v, wsem)
            drain_out()
            drain_out()

    return emb


def kernel(idx, W):
    B, T = idx.shape
    V, D = W.shape
    idxt = idx.T.astype(jnp.int32).reshape(-1)
    wt = W.T
    out3 = _build(T, B, D, V)(idxt, wt)
    return out3.transpose(2, 0, 1)


# speedup vs baseline: 17.3927x; 1.2229x over previous
import functools

import jax
import jax.numpy as jnp
from jax import lax
from jax.experimental import pallas as pl
from jax.experimental.pallas import tpu as pltpu
from jax.experimental.pallas import tpu_sc as plsc

_NC = 2
_NS = 16
_NW = _NC * _NS
_L = 16


def _build(T, B, D, V):
    DPT = D // _NW

    mesh = plsc.VectorSubcoreMesh(core_axis_name="c", subcore_axis_name="s")

    @functools.partial(
        pl.kernel,
        out_type=jax.ShapeDtypeStruct((T, D, B), jnp.float32),
        mesh=mesh,
        scratch_types=[
            pltpu.VMEM((V,), jnp.float32),
            pltpu.VMEM((2 * B,), jnp.int32),
            pltpu.VMEM((2 * B,), jnp.float32),
            pltpu.VMEM_SHARED((T * B,), jnp.int32),
            pltpu.SemaphoreType.DMA,
            pltpu.SemaphoreType.DMA,
            pltpu.SemaphoreType.DMA,
        ],
        compiler_params=pltpu.CompilerParams(needs_layout_passes=False),
    )
    def emb(idxt_hbm, wt_hbm, out_hbm, row_v, idx2_v, out2_v, idx_sh, isem, osem, wsem):
        wid = lax.axis_index("s") * _NC + lax.axis_index("c")
        sid = lax.axis_index("s")

        pltpu.async_copy(wt_hbm.at[wid], row_v, wsem)

        chunk = (T * B) // _NS
        pltpu.sync_copy(
            idxt_hbm.at[pl.ds(sid * chunk, chunk)],
            idx_sh.at[pl.ds(sid * chunk, chunk)],
        )
        plsc.subcore_barrier()

        def drain_idx():
            pltpu.make_async_copy(
                idx_sh.at[pl.ds(0, B)], idx2_v.at[pl.ds(0, B)], isem
            ).wait()

        def drain_out():
            pltpu.make_async_copy(out2_v.at[pl.ds(0, B)], out_hbm.at[0].at[0], osem).wait()

        for di in range(DPT):
            d = wid + di * _NW
            pltpu.async_copy(idx_sh.at[pl.ds(0, B)], idx2_v.at[pl.ds(0, B)], isem)
            pltpu.make_async_copy(wt_hbm.at[d], row_v, wsem).wait()

            def per_t(t, carry):
                iu, pu = carry
                ib = iu * B
                pb = pu * B

                @pl.when(t + 1 < T)
                def _():
                    pltpu.async_copy(
                        idx_sh.at[pl.ds((t + 1) * B, B)],
                        idx2_v.at[pl.ds((1 - iu) * B, B)],
                        isem,
                    )

                drain_idx()

                @pl.when(t >= 2)
                def _():
                    drain_out()

                @plsc.parallel_loop(0, B, step=_L, unroll=32)
                def _(i):
                    out2_v[pl.ds(pb + i, _L)] = plsc.load_gather(
                        row_v, [idx2_v[pl.ds(ib + i, _L)]]
                    )

                pltpu.async_copy(
                    out2_v.at[pl.ds(pb, B)], out_hbm.at[t].at[d], osem
                )
                return (1 - iu, 1 - pu)

            lax.fori_loop(0, T, per_t, (0, 0))
            if di + 1 < DPT:
                pltpu.async_copy(wt_hbm.at[wid + (di + 1) * _NW], row_v, wsem)
            drain_out()
            drain_out()

    return emb


def kernel(idx, W):
    B, T = idx.shape
    V, D = W.shape
    idxt = idx.T.astype(jnp.int32).reshape(-1)
    wt = W.T
    out3 = _build(T, B, D, V)(idxt, wt)
    return out3.transpose(2, 0, 1)
